# baseline (speedup 1.0000x reference)
import functools

import jax
import jax.numpy as jnp
from jax import lax
from jax.experimental import pallas as pl
from jax.experimental.pallas import tpu as pltpu
from jax.experimental.pallas import tpu_sc as plsc

NC = 2
NS = 16
NW = NC * NS
LANES = 16
TILE = 1024
WT = 14
CHW = WT * 2 * TILE


def _mixup_body(nchunks, x_hbm, idx_hbm, w_hbm, out_hbm,
                idx_v, w_v, a0, a1, o0, o1, si0, si1, so0, so1):
    per_w = nchunks // NW
    c = lax.axis_index("c")
    s = lax.axis_index("s")
    wid = s * NC + c
    base = wid * per_w

    pltpu.sync_copy(idx_hbm, idx_v)
    pltpu.sync_copy(w_hbm, w_v)
    wa = w_v[...]
    wb = 1.0 - wa
    bgroups = 256 // LANES
    pvecs = []
    for j in range(bgroups):
        p = idx_v[pl.ds(j * LANES, LANES)]
        pvecs.append(
            lax.shift_left(lax.shift_right_logical(p, 7), 10)
            + lax.bitwise_and(p, 127))

    abufs = (a0, a1)
    obufs = (o0, o1)
    isems = (si0, si1)
    osems = (so0, so1)

    def start_in(t, k):
        pltpu.async_copy(
            x_hbm.at[pl.ds((base + t) * CHW, CHW)], abufs[k], isems[k])

    def wait_in(k):
        pltpu.make_async_copy(
            x_hbm.at[pl.ds(0, CHW)], abufs[k], isems[k]).wait()

    def start_out(t, k):
        pltpu.async_copy(
            obufs[k], out_hbm.at[pl.ds((base + t) * CHW, CHW)], osems[k])

    def wait_out(k):
        pltpu.make_async_copy(
            obufs[k], out_hbm.at[pl.ds(0, CHW)], osems[k]).wait()

    start_in(0, 0)

    def group(g, _):
        for k in (0, 1):
            t = g * 2 + k
            nk = 1 - k

            @pl.when(t + 1 < per_w)
            def _():
                start_in(t + 1, nk)

            wait_in(k)

            @pl.when(t >= 2)
            def _():
                wait_out(k)

            def vec_body(w, _, k=k):
                wt = lax.shift_right_logical(w, 3)
                w8 = lax.bitwise_and(w, 7)
                rb = wt * (2 * TILE) + w8 * 128
                rb = pl.multiple_of(rb, 128)
                for j in range(bgroups):
                    off = (j // 8) * TILE + (j % 8) * LANES
                    a = abufs[k][pl.ds(rb + off, LANES)]
                    b = plsc.load_gather(abufs[k], [pvecs[j] + rb])
                    obufs[k][pl.ds(rb + off, LANES)] = wa * a + wb * b
                return 0

            lax.fori_loop(0, WT * 8, vec_body, 0)
            start_out(t, k)
        return 0

    lax.fori_loop(0, per_w // 2, group, 0)
    wait_out(0)
    wait_out(1)


def kernel(inputs, index, lam):
    B, C, H, W = inputs.shape
    assert B == 256 and W % 8 == 0
    xt = jnp.transpose(inputs, (1, 2, 3, 0))
    x6 = xt.reshape(C, H, W // 8, 8, B // 128, 128)
    xp = jnp.transpose(x6, (0, 1, 2, 4, 3, 5))
    xflat = xp.reshape(-1)

    nchunks = C * H * (W // 8) // WT
    assert nchunks % NW == 0

    idx = index.astype(jnp.int32)
    w = jnp.full((LANES,), lam, dtype=jnp.float32)

    mesh = plsc.VectorSubcoreMesh(
        core_axis_name="c", subcore_axis_name="s",
        num_cores=NC, num_subcores=NS)

    run = pl.kernel(
        functools.partial(_mixup_body, nchunks),
        out_type=jax.ShapeDtypeStruct((xflat.size,), jnp.float32),
        mesh=mesh,
        compiler_params=pltpu.CompilerParams(
            use_tc_tiling_on_sc=False, needs_layout_passes=False),
        scratch_types=[
            pltpu.VMEM((B,), jnp.int32),
            pltpu.VMEM((LANES,), jnp.float32),
            pltpu.VMEM((CHW,), jnp.float32),
            pltpu.VMEM((CHW,), jnp.float32),
            pltpu.VMEM((CHW,), jnp.float32),
            pltpu.VMEM((CHW,), jnp.float32),
            pltpu.SemaphoreType.DMA,
            pltpu.SemaphoreType.DMA,
            pltpu.SemaphoreType.DMA,
            pltpu.SemaphoreType.DMA,
        ],
    )
    outflat = run(xflat, idx, w)
    outp = outflat.reshape(C, H, W // 8, B // 128, 8, 128)
    outt = jnp.transpose(outp, (0, 1, 2, 4, 3, 5)).reshape(C, H, W, B)
    return jnp.transpose(outt, (3, 0, 1, 2))


# speedup vs baseline: 2.0826x; 2.0826x over previous
import functools

import jax
import jax.numpy as jnp
from jax import lax
from jax.experimental import pallas as pl
from jax.experimental.pallas import tpu as pltpu
from jax.experimental.pallas import tpu_sc as plsc

NC = 2
NS = 16
NW = NC * NS
LANES = 16
TILE = 1024
WT = 14
CHW = WT * 2 * TILE


def _mixup_body(nchunks, x_hbm, idx_hbm, w_hbm, out_hbm,
                idx_v, w_v, a0, a1, o0, o1, si0, si1, so0, so1):
    per_w = nchunks // NW
    c = lax.axis_index("c")
    s = lax.axis_index("s")
    wid = s * NC + c
    base = wid * per_w

    pltpu.sync_copy(idx_hbm, idx_v)
    pltpu.sync_copy(w_hbm, w_v)
    wa = w_v[...]
    wb = 1.0 - wa
    bgroups = 256 // LANES
    pvecs = []
    for j in range(bgroups):
        p = idx_v[pl.ds(j * LANES, LANES)]
        pvecs.append(
            lax.shift_left(lax.shift_right_logical(p, 7), 10)
            + lax.bitwise_and(p, 127))

    abufs = (a0, a1)
    obufs = (o0, o1)
    isems = (si0, si1)
    osems = (so0, so1)

    def start_in(t, k):
        pltpu.async_copy(
            x_hbm.at[pl.ds((base + t) * CHW, CHW)], abufs[k], isems[k])

    def wait_in(k):
        pltpu.make_async_copy(
            x_hbm.at[pl.ds(0, CHW)], abufs[k], isems[k]).wait()

    def start_out(t, k):
        pltpu.async_copy(
            obufs[k], out_hbm.at[pl.ds((base + t) * CHW, CHW)], osems[k])

    def wait_out(k):
        pltpu.make_async_copy(
            obufs[k], out_hbm.at[pl.ds(0, CHW)], osems[k]).wait()

    start_in(0, 0)

    def group(g, _):
        for k in (0, 1):
            t = g * 2 + k
            nk = 1 - k

            @pl.when(t + 1 < per_w)
            def _():
                start_in(t + 1, nk)

            wait_in(k)

            @pl.when(t >= 2)
            def _():
                wait_out(k)

            @plsc.parallel_loop(0, WT * 8, 1, unroll=2)
            def _(w, k=k):
                wt = lax.shift_right_logical(w, 3)
                w8 = lax.bitwise_and(w, 7)
                rb = wt * (2 * TILE) + w8 * 128
                rb = pl.multiple_of(rb, 128)
                for j in range(bgroups):
                    off = (j // 8) * TILE + (j % 8) * LANES
                    a = abufs[k][pl.ds(rb + off, LANES)]
                    b = plsc.load_gather(abufs[k], [pvecs[j] + rb])
                    obufs[k][pl.ds(rb + off, LANES)] = wa * a + wb * b
            start_out(t, k)
        return 0

    lax.fori_loop(0, per_w // 2, group, 0)
    wait_out(0)
    wait_out(1)


def kernel(inputs, index, lam):
    B, C, H, W = inputs.shape
    assert B == 256 and W % 8 == 0
    xt = jnp.transpose(inputs, (1, 2, 3, 0))
    x6 = xt.reshape(C, H, W // 8, 8, B // 128, 128)
    xp = jnp.transpose(x6, (0, 1, 2, 4, 3, 5))
    xflat = xp.reshape(-1)

    nchunks = C * H * (W // 8) // WT
    assert nchunks % NW == 0

    idx = index.astype(jnp.int32)
    w = jnp.full((LANES,), lam, dtype=jnp.float32)

    mesh = plsc.VectorSubcoreMesh(
        core_axis_name="c", subcore_axis_name="s",
        num_cores=NC, num_subcores=NS)

    run = pl.kernel(
        functools.partial(_mixup_body, nchunks),
        out_type=jax.ShapeDtypeStruct((xflat.size,), jnp.float32),
        mesh=mesh,
        compiler_params=pltpu.CompilerParams(
            use_tc_tiling_on_sc=False, needs_layout_passes=False),
        scratch_types=[
            pltpu.VMEM((B,), jnp.int32),
            pltpu.VMEM((LANES,), jnp.float32),
            pltpu.VMEM((CHW,), jnp.float32),
            pltpu.VMEM((CHW,), jnp.float32),
            pltpu.VMEM((CHW,), jnp.float32),
            pltpu.VMEM((CHW,), jnp.float32),
            pltpu.SemaphoreType.DMA,
            pltpu.SemaphoreType.DMA,
            pltpu.SemaphoreType.DMA,
            pltpu.SemaphoreType.DMA,
        ],
    )
    outflat = run(xflat, idx, w)
    outp = outflat.reshape(C, H, W // 8, B // 128, 8, 128)
    outt = jnp.transpose(outp, (0, 1, 2, 4, 3, 5)).reshape(C, H, W, B)
    return jnp.transpose(outt, (3, 0, 1, 2))


# speedup vs baseline: 2.4515x; 1.1771x over previous
import functools

import jax
import jax.numpy as jnp
from jax import lax
from jax.experimental import pallas as pl
from jax.experimental.pallas import tpu as pltpu
from jax.experimental.pallas import tpu_sc as plsc

NC = 2
NS = 16
NW = NC * NS
LANES = 16
TILE = 1024
WT = 14
CHW = WT * 2 * TILE


def _mixup_body(nchunks, x_hbm, idx_hbm, w_hbm, out_hbm,
                idx_v, w_v, a0, a1, o0, o1, si0, si1, so0, so1):
    per_w = nchunks // NW
    c = lax.axis_index("c")
    s = lax.axis_index("s")
    wid = s * NC + c
    base = wid * per_w

    pltpu.sync_copy(idx_hbm, idx_v)
    pltpu.sync_copy(w_hbm, w_v)
    wa = w_v[...]
    wb = 1.0 - wa
    bgroups = 256 // LANES
    pvecs = []
    for j in range(bgroups):
        p = idx_v[pl.ds(j * LANES, LANES)]
        pvecs.append(
            lax.shift_left(lax.shift_right_logical(p, 7), 10)
            + lax.bitwise_and(p, 127))

    abufs = (a0, a1)
    obufs = (o0, o1)
    isems = (si0, si1)
    osems = (so0, so1)

    def start_in(t, k):
        pltpu.async_copy(
            x_hbm.at[pl.ds((base + t) * CHW, CHW)], abufs[k], isems[k])

    def wait_in(k):
        pltpu.make_async_copy(
            x_hbm.at[pl.ds(0, CHW)], abufs[k], isems[k]).wait()

    def start_out(t, k):
        pltpu.async_copy(
            obufs[k], out_hbm.at[pl.ds((base + t) * CHW, CHW)], osems[k])

    def wait_out(k):
        pltpu.make_async_copy(
            obufs[k], out_hbm.at[pl.ds(0, CHW)], osems[k]).wait()

    start_in(0, 0)

    def group(g, _):
        for k in (0, 1):
            t = g * 2 + k
            nk = 1 - k

            @pl.when(t + 1 < per_w)
            def _():
                start_in(t + 1, nk)

            wait_in(k)

            @pl.when(t >= 2)
            def _():
                wait_out(k)

            @plsc.parallel_loop(0, WT * 8, 1, unroll=4)
            def _(w, k=k):
                wt = lax.shift_right_logical(w, 3)
                w8 = lax.bitwise_and(w, 7)
                rb = wt * (2 * TILE) + w8 * 128
                rb = pl.multiple_of(rb, 128)
                for j in range(bgroups):
                    off = (j // 8) * TILE + (j % 8) * LANES
                    a = abufs[k][pl.ds(rb + off, LANES)]
                    b = plsc.load_gather(abufs[k], [pvecs[j] + rb])
                    obufs[k][pl.ds(rb + off, LANES)] = wa * a + wb * b
            start_out(t, k)
        return 0

    lax.fori_loop(0, per_w // 2, group, 0)
    wait_out(0)
    wait_out(1)


def kernel(inputs, index, lam):
    B, C, H, W = inputs.shape
    assert B == 256 and W % 8 == 0
    xt = jnp.transpose(inputs, (1, 2, 3, 0))
    x6 = xt.reshape(C, H, W // 8, 8, B // 128, 128)
    xp = jnp.transpose(x6, (0, 1, 2, 4, 3, 5))
    xflat = xp.reshape(-1)

    nchunks = C * H * (W // 8) // WT
    assert nchunks % NW == 0

    idx = index.astype(jnp.int32)
    w = jnp.full((LANES,), lam, dtype=jnp.float32)

    mesh = plsc.VectorSubcoreMesh(
        core_axis_name="c", subcore_axis_name="s",
        num_cores=NC, num_subcores=NS)

    run = pl.kernel(
        functools.partial(_mixup_body, nchunks),
        out_type=jax.ShapeDtypeStruct((xflat.size,), jnp.float32),
        mesh=mesh,
        compiler_params=pltpu.CompilerParams(
            use_tc_tiling_on_sc=False, needs_layout_passes=False),
        scratch_types=[
            pltpu.VMEM((B,), jnp.int32),
            pltpu.VMEM((LANES,), jnp.float32),
            pltpu.VMEM((CHW,), jnp.float32),
            pltpu.VMEM((CHW,), jnp.float32),
            pltpu.VMEM((CHW,), jnp.float32),
            pltpu.VMEM((CHW,), jnp.float32),
            pltpu.SemaphoreType.DMA,
            pltpu.SemaphoreType.DMA,
            pltpu.SemaphoreType.DMA,
            pltpu.SemaphoreType.DMA,
        ],
    )
    outflat = run(xflat, idx, w)
    outp = outflat.reshape(C, H, W // 8, B // 128, 8, 128)
    outt = jnp.transpose(outp, (0, 1, 2, 4, 3, 5)).reshape(C, H, W, B)
    return jnp.transpose(outt, (3, 0, 1, 2))


# speedup vs baseline: 3.1518x; 1.2857x over previous
import functools

import jax
import jax.numpy as jnp
from jax import lax
from jax.experimental import pallas as pl
from jax.experimental.pallas import tpu as pltpu
from jax.experimental.pallas import tpu_sc as plsc

NC = 2
NS = 16
NW = NC * NS
LANES = 16
TILE = 1024
WT = 14
CHW = WT * 2 * TILE


def _mixup_body(nchunks, x_hbm, idx_hbm, w_hbm, out_hbm,
                idx_v, w_v, a0, a1, o0, o1, si0, si1, so0, so1):
    per_w = nchunks // NW
    c = lax.axis_index("c")
    s = lax.axis_index("s")
    wid = s * NC + c
    base = wid * per_w

    pltpu.sync_copy(idx_hbm, idx_v)
    pltpu.sync_copy(w_hbm, w_v)
    wa = w_v[...]
    wb = 1.0 - wa
    bgroups = 256 // LANES
    pvecs = []
    for j in range(bgroups):
        p = idx_v[pl.ds(j * LANES, LANES)]
        pvecs.append(
            lax.shift_left(lax.shift_right_logical(p, 7), 10)
            + lax.bitwise_and(p, 127))

    abufs = (a0, a1)
    obufs = (o0, o1)
    isems = (si0, si1)
    osems = (so0, so1)

    def start_in(t, k):
        pltpu.async_copy(
            x_hbm.at[pl.ds((base + t) * CHW, CHW)], abufs[k], isems[k])

    def wait_in(k):
        pltpu.make_async_copy(
            x_hbm.at[pl.ds(0, CHW)], abufs[k], isems[k]).wait()

    def start_out(t, k):
        pltpu.async_copy(
            obufs[k], out_hbm.at[pl.ds((base + t) * CHW, CHW)], osems[k])

    def wait_out(k):
        pltpu.make_async_copy(
            obufs[k], out_hbm.at[pl.ds(0, CHW)], osems[k]).wait()

    start_in(0, 0)

    def group(g, _):
        for k in (0, 1):
            t = g * 2 + k
            nk = 1 - k

            @pl.when(t + 1 < per_w)
            def _():
                start_in(t + 1, nk)

            wait_in(k)

            @pl.when(t >= 2)
            def _():
                wait_out(k)

            @plsc.parallel_loop(0, WT * 8, 1, unroll=8)
            def _(w, k=k):
                wt = lax.shift_right_logical(w, 3)
                w8 = lax.bitwise_and(w, 7)
                rb = wt * (2 * TILE) + w8 * 128
                rb = pl.multiple_of(rb, 128)
                for j in range(bgroups):
                    off = (j // 8) * TILE + (j % 8) * LANES
                    a = abufs[k][pl.ds(rb + off, LANES)]
                    b = plsc.load_gather(abufs[k], [pvecs[j] + rb])
                    obufs[k][pl.ds(rb + off, LANES)] = wa * a + wb * b
            start_out(t, k)
        return 0

    lax.fori_loop(0, per_w // 2, group, 0)
    wait_out(0)
    wait_out(1)


def kernel(inputs, index, lam):
    B, C, H, W = inputs.shape
    assert B == 256 and W % 8 == 0
    xt = jnp.transpose(inputs, (1, 2, 3, 0))
    x6 = xt.reshape(C, H, W // 8, 8, B // 128, 128)
    xp = jnp.transpose(x6, (0, 1, 2, 4, 3, 5))
    xflat = xp.reshape(-1)

    nchunks = C * H * (W // 8) // WT
    assert nchunks % NW == 0

    idx = index.astype(jnp.int32)
    w = jnp.full((LANES,), lam, dtype=jnp.float32)

    mesh = plsc.VectorSubcoreMesh(
        core_axis_name="c", subcore_axis_name="s",
        num_cores=NC, num_subcores=NS)

    run = pl.kernel(
        functools.partial(_mixup_body, nchunks),
        out_type=jax.ShapeDtypeStruct((xflat.size,), jnp.float32),
        mesh=mesh,
        compiler_params=pltpu.CompilerParams(
            use_tc_tiling_on_sc=False, needs_layout_passes=False),
        scratch_types=[
            pltpu.VMEM((B,), jnp.int32),
            pltpu.VMEM((LANES,), jnp.float32),
            pltpu.VMEM((CHW,), jnp.float32),
            pltpu.VMEM((CHW,), jnp.float32),
            pltpu.VMEM((CHW,), jnp.float32),
            pltpu.VMEM((CHW,), jnp.float32),
            pltpu.SemaphoreType.DMA,
            pltpu.SemaphoreType.DMA,
            pltpu.SemaphoreType.DMA,
            pltpu.SemaphoreType.DMA,
        ],
    )
    outflat = run(xflat, idx, w)
    outp = outflat.reshape(C, H, W // 8, B // 128, 8, 128)
    outt = jnp.transpose(outp, (0, 1, 2, 4, 3, 5)).reshape(C, H, W, B)
    return jnp.transpose(outt, (3, 0, 1, 2))


# speedup vs baseline: 3.2846x; 1.0421x over previous
import functools

import jax
import jax.numpy as jnp
from jax import lax
from jax.experimental import pallas as pl
from jax.experimental.pallas import tpu as pltpu
from jax.experimental.pallas import tpu_sc as plsc

NC = 2
NS = 16
NW = NC * NS
LANES = 16
TILE = 1024
WT = 14
CHW = WT * 2 * TILE


def _mixup_body(nchunks, x_hbm, idx_hbm, w_hbm, out_hbm,
                idx_v, w_v, a0, a1, o0, o1, si0, si1, so0, so1):
    per_w = nchunks // NW
    c = lax.axis_index("c")
    s = lax.axis_index("s")
    wid = s * NC + c
    base = wid * per_w

    pltpu.sync_copy(idx_hbm, idx_v)
    pltpu.sync_copy(w_hbm, w_v)
    wa = w_v[...]
    wb = 1.0 - wa
    bgroups = 256 // LANES
    pvecs = []
    for j in range(bgroups):
        p = idx_v[pl.ds(j * LANES, LANES)]
        pvecs.append(
            lax.shift_left(lax.shift_right_logical(p, 7), 10)
            + lax.bitwise_and(p, 127))

    abufs = (a0, a1)
    obufs = (o0, o1)
    isems = (si0, si1)
    osems = (so0, so1)

    def start_in(t, k):
        pltpu.async_copy(
            x_hbm.at[pl.ds((base + t) * CHW, CHW)], abufs[k], isems[k])

    def wait_in(k):
        pltpu.make_async_copy(
            x_hbm.at[pl.ds(0, CHW)], abufs[k], isems[k]).wait()

    def start_out(t, k):
        pltpu.async_copy(
            obufs[k], out_hbm.at[pl.ds((base + t) * CHW, CHW)], osems[k])

    def wait_out(k):
        pltpu.make_async_copy(
            obufs[k], out_hbm.at[pl.ds(0, CHW)], osems[k]).wait()

    start_in(0, 0)

    def group(g, _):
        for k in (0, 1):
            t = g * 2 + k
            nk = 1 - k

            @pl.when(t + 1 < per_w)
            def _():
                start_in(t + 1, nk)

            wait_in(k)

            @pl.when(t >= 2)
            def _():
                wait_out(k)

            @plsc.parallel_loop(0, WT * 8, 1, unroll=16)
            def _(w, k=k):
                wt = lax.shift_right_logical(w, 3)
                w8 = lax.bitwise_and(w, 7)
                rb = wt * (2 * TILE) + w8 * 128
                rb = pl.multiple_of(rb, 128)
                for j in range(bgroups):
                    off = (j // 8) * TILE + (j % 8) * LANES
                    a = abufs[k][pl.ds(rb + off, LANES)]
                    b = plsc.load_gather(abufs[k], [pvecs[j] + rb])
                    obufs[k][pl.ds(rb + off, LANES)] = wa * a + wb * b
            start_out(t, k)
        return 0

    lax.fori_loop(0, per_w // 2, group, 0)
    wait_out(0)
    wait_out(1)


def kernel(inputs, index, lam):
    B, C, H, W = inputs.shape
    assert B == 256 and W % 8 == 0
    xt = jnp.transpose(inputs, (1, 2, 3, 0))
    x6 = xt.reshape(C, H, W // 8, 8, B // 128, 128)
    xp = jnp.transpose(x6, (0, 1, 2, 4, 3, 5))
    xflat = xp.reshape(-1)

    nchunks = C * H * (W // 8) // WT
    assert nchunks % NW == 0

    idx = index.astype(jnp.int32)
    w = jnp.full((LANES,), lam, dtype=jnp.float32)

    mesh = plsc.VectorSubcoreMesh(
        core_axis_name="c", subcore_axis_name="s",
        num_cores=NC, num_subcores=NS)

    run = pl.kernel(
        functools.partial(_mixup_body, nchunks),
        out_type=jax.ShapeDtypeStruct((xflat.size,), jnp.float32),
        mesh=mesh,
        compiler_params=pltpu.CompilerParams(
            use_tc_tiling_on_sc=False, needs_layout_passes=False),
        scratch_types=[
            pltpu.VMEM((B,), jnp.int32),
            pltpu.VMEM((LANES,), jnp.float32),
            pltpu.VMEM((CHW,), jnp.float32),
            pltpu.VMEM((CHW,), jnp.float32),
            pltpu.VMEM((CHW,), jnp.float32),
            pltpu.VMEM((CHW,), jnp.float32),
            pltpu.SemaphoreType.DMA,
            pltpu.SemaphoreType.DMA,
            pltpu.SemaphoreType.DMA,
            pltpu.SemaphoreType.DMA,
        ],
    )
    outflat = run(xflat, idx, w)
    outp = outflat.reshape(C, H, W // 8, B // 128, 8, 128)
    outt = jnp.transpose(outp, (0, 1, 2, 4, 3, 5)).reshape(C, H, W, B)
    return jnp.transpose(outt, (3, 0, 1, 2))
